# Initial kernel scaffold; baseline (speedup 1.0000x reference)
import jax
import jax.numpy as jnp
from jax.experimental import pallas as pl


def kernel(x, W):
    raise NotImplementedError("write your pallas kernel here")


# speedup vs baseline: 1.3444x; 1.3444x over previous
import functools

import jax
import jax.numpy as jnp
from jax.experimental import pallas as pl
from jax.experimental.pallas import tpu as pltpu

K = 8


def _router_kernel(x_ref, w_ref, gates_ref, idx_ref, loss_ref, imp_ref,
                   *, num_blocks: int):
    i = pl.program_id(0)

    logits = jax.lax.dot_general(
        x_ref[...], w_ref[...],
        dimension_numbers=(((1,), (1,)), ((), ())),
        preferred_element_type=jnp.float32,
    )

    m = jnp.max(logits, axis=1, keepdims=True)
    e = jnp.exp(logits - m)
    s = jnp.sum(e, axis=1, keepdims=True)
    probs = e / s

    @pl.when(i == 0)
    def _init():
        imp_ref[...] = jnp.zeros_like(imp_ref)

    imp_ref[...] += jnp.sum(probs, axis=0, keepdims=True)

    tb, e_dim = probs.shape
    lane = jax.lax.broadcasted_iota(jnp.int32, (tb, e_dim), 1)
    g = probs
    vals = []
    idxs = []
    for _ in range(K):
        v = jnp.max(g, axis=1, keepdims=True)
        ix = jnp.argmax(g, axis=1).astype(jnp.int32)
        vals.append(v)
        idxs.append(ix[:, None])
        g = jnp.where(lane == ix[:, None], -jnp.inf, g)
    gates_ref[...] = jnp.concatenate(vals, axis=1)
    idx_ref[...] = jnp.concatenate(idxs, axis=1)

    @pl.when(i == num_blocks - 1)
    def _finish():
        imp = imp_ref[...]
        mean = jnp.mean(imp)
        var = jnp.mean((imp - mean) ** 2)
        loss_ref[...] = jnp.reshape(var / (mean + 1e-6) ** 2, (1, 1))


def kernel(x, W):
    T, D = x.shape
    E = W.shape[0]
    TB = 512
    num_blocks = T // TB

    gates, idx, loss = pl.pallas_call(
        functools.partial(_router_kernel, num_blocks=num_blocks),
        grid=(num_blocks,),
        in_specs=[
            pl.BlockSpec((TB, D), lambda i: (i, 0)),
            pl.BlockSpec((E, D), lambda i: (0, 0)),
        ],
        out_specs=[
            pl.BlockSpec((TB, K), lambda i: (i, 0)),
            pl.BlockSpec((TB, K), lambda i: (i, 0)),
            pl.BlockSpec((1, 1), lambda i: (0, 0)),
        ],
        out_shape=[
            jax.ShapeDtypeStruct((T, K), jnp.float32),
            jax.ShapeDtypeStruct((T, K), jnp.int32),
            jax.ShapeDtypeStruct((1, 1), jnp.float32),
        ],
        scratch_shapes=[pltpu.VMEM((1, E), jnp.float32)],
    )(x, W)

    return gates, idx, loss.reshape(())


# speedup vs baseline: 1.5092x; 1.1226x over previous
import functools

import jax
import jax.numpy as jnp
from jax.experimental import pallas as pl
from jax.experimental.pallas import tpu as pltpu

K = 8


def _router_kernel(x_ref, w_ref, gates_ref, idx_ref, loss_ref, imp_ref,
                   *, num_blocks: int):
    i = pl.program_id(0)

    logits = jax.lax.dot_general(
        x_ref[...], w_ref[...],
        dimension_numbers=(((1,), (1,)), ((), ())),
        preferred_element_type=jnp.float32,
    )

    m = jnp.max(logits, axis=1, keepdims=True)
    e = jnp.exp(logits - m)
    s = jnp.sum(e, axis=1, keepdims=True)
    probs = e / s

    @pl.when(i == 0)
    def _init():
        imp_ref[...] = jnp.zeros_like(imp_ref)

    imp_ref[...] += jnp.sum(probs, axis=0, keepdims=True)

    tb, e_dim = probs.shape
    lane = jax.lax.broadcasted_iota(jnp.int32, (tb, e_dim), 1)
    g = probs
    vals = []
    idxs = []
    for _ in range(K):
        v = jnp.max(g, axis=1, keepdims=True)
        ix = jnp.argmax(g, axis=1).astype(jnp.int32)
        vals.append(v)
        idxs.append(ix[:, None])
        g = jnp.where(lane == ix[:, None], -jnp.inf, g)
    gates_ref[...] = jnp.concatenate(vals, axis=1)
    idx_ref[...] = jnp.concatenate(idxs, axis=1)

    @pl.when(i == num_blocks - 1)
    def _finish():
        imp = imp_ref[...]
        mean = jnp.mean(imp)
        var = jnp.mean((imp - mean) ** 2)
        loss_ref[...] = jnp.reshape(var / (mean + 1e-6) ** 2, (1, 1))


def kernel(x, W):
    T, D = x.shape
    E = W.shape[0]
    TB = 1024
    num_blocks = T // TB

    gates, idx, loss = pl.pallas_call(
        functools.partial(_router_kernel, num_blocks=num_blocks),
        grid=(num_blocks,),
        in_specs=[
            pl.BlockSpec((TB, D), lambda i: (i, 0)),
            pl.BlockSpec((E, D), lambda i: (0, 0)),
        ],
        out_specs=[
            pl.BlockSpec((TB, K), lambda i: (i, 0)),
            pl.BlockSpec((TB, K), lambda i: (i, 0)),
            pl.BlockSpec((1, 1), lambda i: (0, 0)),
        ],
        out_shape=[
            jax.ShapeDtypeStruct((T, K), jnp.float32),
            jax.ShapeDtypeStruct((T, K), jnp.int32),
            jax.ShapeDtypeStruct((1, 1), jnp.float32),
        ],
        scratch_shapes=[pltpu.VMEM((1, E), jnp.float32)],
    )(x, W)

    return gates, idx, loss.reshape(())


# speedup vs baseline: 1.5152x; 1.0040x over previous
import functools

import jax
import jax.numpy as jnp
from jax.experimental import pallas as pl
from jax.experimental.pallas import tpu as pltpu

K = 8


def _router_kernel(x_ref, w_ref, gates_ref, idx_ref, loss_ref, imp_ref,
                   *, num_blocks: int):
    i = pl.program_id(0)

    logits = jax.lax.dot_general(
        x_ref[...], w_ref[...],
        dimension_numbers=(((1,), (1,)), ((), ())),
        preferred_element_type=jnp.float32,
    )

    m = jnp.max(logits, axis=1, keepdims=True)
    e = jnp.exp(logits - m)
    s = jnp.sum(e, axis=1, keepdims=True)
    probs = e / s

    @pl.when(i == 0)
    def _init():
        imp_ref[...] = jnp.zeros_like(imp_ref)

    imp_ref[...] += jnp.sum(probs, axis=0, keepdims=True)

    tb, e_dim = probs.shape
    lane = jax.lax.broadcasted_iota(jnp.int32, (tb, e_dim), 1)
    g = probs
    vals = []
    idxs = []
    for _ in range(K):
        v = jnp.max(g, axis=1, keepdims=True)
        ix = jnp.argmax(g, axis=1).astype(jnp.int32)
        vals.append(v)
        idxs.append(ix[:, None])
        g = jnp.where(lane == ix[:, None], -jnp.inf, g)
    gates_ref[...] = jnp.concatenate(vals, axis=1)
    idx_ref[...] = jnp.concatenate(idxs, axis=1)

    @pl.when(i == num_blocks - 1)
    def _finish():
        imp = imp_ref[...]
        mean = jnp.mean(imp)
        var = jnp.mean((imp - mean) ** 2)
        loss_ref[...] = jnp.reshape(var / (mean + 1e-6) ** 2, (1, 1))


def kernel(x, W):
    T, D = x.shape
    E = W.shape[0]
    TB = 1024
    num_blocks = T // TB

    gates, idx, loss = pl.pallas_call(
        functools.partial(_router_kernel, num_blocks=num_blocks),
        grid=(num_blocks,),
        in_specs=[
            pl.BlockSpec((TB, D), lambda i: (i, 0)),
            pl.BlockSpec((E, D), lambda i: (0, 0)),
        ],
        out_specs=[
            pl.BlockSpec((TB, K), lambda i: (i, 0)),
            pl.BlockSpec((TB, K), lambda i: (i, 0)),
            pl.BlockSpec((1, 1), lambda i: (0, 0)),
        ],
        out_shape=[
            jax.ShapeDtypeStruct((T, K), jnp.float32),
            jax.ShapeDtypeStruct((T, K), jnp.int32),
            jax.ShapeDtypeStruct((1, 1), jnp.float32),
        ],
        scratch_shapes=[pltpu.VMEM((1, E), jnp.float32)],
        compiler_params=pltpu.CompilerParams(
            vmem_limit_bytes=120 * 1024 * 1024,
        ),
    )(x, W)

    return gates, idx, loss.reshape(())
